# baseline (speedup 1.0000x reference)
import functools

import jax
import jax.numpy as jnp
from jax import lax
from jax.experimental import pallas as pl
from jax.experimental.pallas import tpu as pltpu
from jax.experimental.pallas import tpu_sc as plsc

NW = 32
P = 64
LANES = 16


@functools.lru_cache(maxsize=None)
def _transpose_call(C, H, W, HB):
    Ch = C // 2

    def body(in_ref, out_ref):
        blk = in_ref[0]
        t = jnp.reshape(blk, (C, HB * W)).T
        a = t[:, :Ch].astype(jnp.bfloat16).astype(jnp.float32)
        b = t[:, Ch:].astype(jnp.bfloat16).astype(jnp.float32)
        au = lax.bitcast_convert_type(a, jnp.int32)
        bu = lax.bitcast_convert_type(b, jnp.int32)
        out_ref[...] = lax.shift_right_logical(au, 16) | bu

    return pl.pallas_call(
        body,
        grid=(H // HB,),
        in_specs=[pl.BlockSpec((1, C, HB, W), lambda i: (0, 0, i, 0))],
        out_specs=pl.BlockSpec((HB * W, Ch), lambda i: (i, 0)),
        out_shape=jax.ShapeDtypeStruct((H * W, Ch), jnp.int32),
    )


@functools.lru_cache(maxsize=None)
def _sc_sample_call(N, C, H, W, Nv, K):
    rows_per_tile = N // NW
    nchunks = rows_per_tile // P
    mesh = plsc.VectorSubcoreMesh(core_axis_name="c", subcore_axis_name="s")

    @functools.partial(
        pl.kernel,
        mesh=mesh,
        out_type=jax.ShapeDtypeStruct((N, C), jnp.float32),
        compiler_params=pltpu.CompilerParams(
            needs_layout_passes=False, use_tc_tiling_on_sc=False),
        scratch_types=[
            pltpu.VMEM((Nv * 2,), jnp.float32),
            pltpu.VMEM((K * 3,), jnp.float32),
            pltpu.VMEM((rows_per_tile * 3,), jnp.int32),
            pltpu.VMEM((2, 4, P), jnp.int32),
            pltpu.VMEM((2, 4, P), jnp.float32),
            pltpu.VMEM((2, 4, P, C // 2), jnp.int32),
            pltpu.VMEM((2, P, C), jnp.float32),
            pltpu.SemaphoreType.DMA,
            pltpu.SemaphoreType.DMA,
            pltpu.SemaphoreType.DMA,
            pltpu.SemaphoreType.DMA,
        ],
    )
    def sc_fn(table, parents_hbm, bary_hbm, v2d_hbm, out_hbm,
              v2d_v, bary_v, par_v, idx_v, w_v, rows_v, out_v,
              gsem0, gsem1, osem0, osem1):
        cid = lax.axis_index("c")
        sid = lax.axis_index("s")
        wid = sid * 2 + cid
        base = wid * rows_per_tile
        gsem = (gsem0, gsem1)
        osem = (osem0, osem1)

        pltpu.sync_copy(v2d_hbm, v2d_v)
        pltpu.sync_copy(bary_hbm, bary_v)
        pltpu.sync_copy(parents_hbm.at[pl.ds(base * 3, rows_per_tile * 3)],
                        par_v)

        lane = lax.iota(jnp.int32, LANES)

        def stage_a(ci, b):
            for g in range(P // LANES):
                rloc = ci * P + g * LANES + lane
                n = base + rloc
                k3 = lax.rem(n, K) * 3
                b0 = plsc.load_gather(bary_v, [k3])
                b1 = plsc.load_gather(bary_v, [k3 + 1])
                b2 = plsc.load_gather(bary_v, [k3 + 2])
                r3 = rloc * 3
                p0 = plsc.load_gather(par_v, [r3]) * 2
                p1 = plsc.load_gather(par_v, [r3 + 1]) * 2
                p2 = plsc.load_gather(par_v, [r3 + 2]) * 2
                vx0 = plsc.load_gather(v2d_v, [p0])
                vy0 = plsc.load_gather(v2d_v, [p0 + 1])
                vx1 = plsc.load_gather(v2d_v, [p1])
                vy1 = plsc.load_gather(v2d_v, [p1 + 1])
                vx2 = plsc.load_gather(v2d_v, [p2])
                vy2 = plsc.load_gather(v2d_v, [p2 + 1])
                cx = vx0 * b0 + vx1 * b1 + vx2 * b2
                cy = vy0 * b0 + vy1 * b1 + vy2 * b2
                xn = cx / (W - 1) * 2.0 - 1.0
                yn = cy / (H - 1) * 2.0 - 1.0
                xf = (xn + 1.0) * 0.5 * (W - 1)
                yf = (yn + 1.0) * 0.5 * (H - 1)
                x0 = xf.astype(jnp.int32)
                y0 = yf.astype(jnp.int32)
                lx = xf - x0.astype(jnp.float32)
                ly = yf - y0.astype(jnp.float32)
                x1 = x0 + 1
                y1 = y0 + 1
                mx = x1 < W
                my = y1 < H
                w00 = (1.0 - lx) * (1.0 - ly)
                w01 = jnp.where(mx, lx * (1.0 - ly), 0.0)
                w10 = jnp.where(my, (1.0 - lx) * ly, 0.0)
                w11 = jnp.where(mx & my, lx * ly, 0.0)
                x1c = jnp.minimum(x1, W - 1)
                y1c = jnp.minimum(y1, H - 1)
                yb0 = y0 * W
                yb1 = y1c * W
                s = pl.ds(g * LANES, LANES)
                idx_v[b, 0, s] = yb0 + x0
                idx_v[b, 1, s] = yb0 + x1c
                idx_v[b, 2, s] = yb1 + x0
                idx_v[b, 3, s] = yb1 + x1c
                w_v[b, 0, s] = w00
                w_v[b, 1, s] = w01
                w_v[b, 2, s] = w10
                w_v[b, 3, s] = w11

        def issue_gathers(b):
            for t in range(4):
                pltpu.async_copy(table.at[idx_v.at[b, t]], rows_v.at[b, t],
                                 gsem[b])

        def drain_gathers(b):
            for t in range(4):
                pltpu.make_async_copy(table.at[idx_v.at[b, t]],
                                      rows_v.at[b, t], gsem[b]).wait()

        def combine(ci, b):
            def grp_body(g2, c):
                b16 = g2 * LANES
                sw = pl.ds(b16, LANES)
                wv0 = w_v[b, 0, sw]
                wv1 = w_v[b, 1, sw]
                wv2 = w_v[b, 2, sw]
                wv3 = w_v[b, 3, sw]
                for ps in range(LANES):
                    p = b16 + ps
                    ws = (wv0[ps], wv1[ps], wv2[ps], wv3[ps])
                    for cc in range(C // (2 * LANES)):
                        sl = pl.ds(cc * LANES, LANES)
                        accA = accB = None
                        for t in range(4):
                            pk = plsc.bitcast(rows_v[b, t, p, sl],
                                              jnp.bfloat16)
                            lo, hi = plsc.unpack(
                                pk, format=plsc.PackFormat.INTERLEAVED)
                            if t == 0:
                                accA = lo * ws[t]
                                accB = hi * ws[t]
                            else:
                                accA = accA + lo * ws[t]
                                accB = accB + hi * ws[t]
                        out_v[b, p, pl.ds(cc * LANES, LANES)] = accA
                        out_v[b, p, pl.ds(C // 2 + cc * LANES, LANES)] = accB
                return c

            lax.fori_loop(0, P // LANES, grp_body, 0)

        def out_slice(ci):
            return out_hbm.at[pl.ds(base + ci * P, P)]

        stage_a(0, 0)
        issue_gathers(0)

        def outer(co, carry):
            for b in range(2):
                ci = co * 2 + b
                nb = 1 - b

                @pl.when(ci < nchunks - 1)
                def _():
                    stage_a(ci + 1, nb)
                    issue_gathers(nb)

                drain_gathers(b)

                @pl.when(co > 0)
                def _():
                    pltpu.make_async_copy(out_v.at[b], out_slice(ci - 2),
                                          osem[b]).wait()

                combine(ci, b)
                pltpu.make_async_copy(out_v.at[b], out_slice(ci),
                                      osem[b]).start()
            return carry

        lax.fori_loop(0, nchunks // 2, outer, 0)
        pltpu.make_async_copy(out_v.at[0], out_slice(nchunks - 2),
                              osem[0]).wait()
        pltpu.make_async_copy(out_v.at[1], out_slice(nchunks - 1),
                              osem[1]).wait()

    return sc_fn


def kernel(feature_map, vertices2d, bary, parents):
    B, C, H, W = feature_map.shape
    N = parents.shape[0]
    Nv = vertices2d.shape[0]
    K = bary.shape[0]
    table = _transpose_call(C, H, W, 8)(feature_map)
    out = _sc_sample_call(N, C, H, W, Nv, K)(
        table, parents.reshape(-1), bary.reshape(-1), vertices2d.reshape(-1))
    return out.reshape(B, N, C)


# speedup vs baseline: 1.6395x; 1.6395x over previous
import functools

import jax
import jax.numpy as jnp
from jax import lax
from jax.experimental import pallas as pl
from jax.experimental.pallas import tpu as pltpu
from jax.experimental.pallas import tpu_sc as plsc

NW = 32
P = 64
LANES = 16


@functools.lru_cache(maxsize=None)
def _transpose_call(C, H, W, HB):
    Ch = C // 2

    def body(in_ref, out_ref):
        blk = in_ref[0]
        t = jnp.reshape(blk, (C, HB * W)).T
        a = t[:, :Ch].astype(jnp.bfloat16).astype(jnp.float32)
        b = t[:, Ch:].astype(jnp.bfloat16).astype(jnp.float32)
        au = lax.bitcast_convert_type(a, jnp.int32)
        bu = lax.bitcast_convert_type(b, jnp.int32)
        word = lax.shift_right_logical(au, 16) | bu
        nxt = jnp.concatenate([word[1:], word[:1]], axis=0)
        out_ref[...] = jnp.concatenate([word, nxt], axis=1)

    return pl.pallas_call(
        body,
        grid=(H // HB,),
        in_specs=[pl.BlockSpec((1, C, HB, W), lambda i: (0, 0, i, 0))],
        out_specs=pl.BlockSpec((HB * W, C), lambda i: (i, 0)),
        out_shape=jax.ShapeDtypeStruct((H * W, C), jnp.int32),
    )


@functools.lru_cache(maxsize=None)
def _sc_sample_call(N, C, H, W, Nv, K):
    rows_per_tile = N // NW
    nchunks = rows_per_tile // P
    mesh = plsc.VectorSubcoreMesh(core_axis_name="c", subcore_axis_name="s")

    @functools.partial(
        pl.kernel,
        mesh=mesh,
        out_type=jax.ShapeDtypeStruct((N, C), jnp.float32),
        compiler_params=pltpu.CompilerParams(needs_layout_passes=False),
        scratch_types=[
            pltpu.VMEM((Nv * 2,), jnp.float32),
            pltpu.VMEM((K * 3,), jnp.float32),
            pltpu.VMEM((rows_per_tile * 3,), jnp.int32),
            pltpu.VMEM((2, 2, P), jnp.int32),
            pltpu.VMEM((2, 4, P), jnp.float32),
            pltpu.VMEM((2, 2, P, C), jnp.int32),
            pltpu.VMEM((2, P, C), jnp.float32),
            pltpu.SemaphoreType.DMA,
            pltpu.SemaphoreType.DMA,
            pltpu.SemaphoreType.DMA,
            pltpu.SemaphoreType.DMA,
        ],
    )
    def sc_fn(table, parents_hbm, bary_hbm, v2d_hbm, out_hbm,
              v2d_v, bary_v, par_v, idx_v, w_v, rows_v, out_v,
              gsem0, gsem1, osem0, osem1):
        cid = lax.axis_index("c")
        sid = lax.axis_index("s")
        wid = sid * 2 + cid
        base = wid * rows_per_tile
        gsem = (gsem0, gsem1)
        osem = (osem0, osem1)

        pltpu.sync_copy(v2d_hbm, v2d_v)
        pltpu.sync_copy(bary_hbm, bary_v)
        pltpu.sync_copy(parents_hbm.at[pl.ds(base * 3, rows_per_tile * 3)],
                        par_v)

        lane = lax.iota(jnp.int32, LANES)

        def stage_a(ci, b):
            for g in range(P // LANES):
                rloc = ci * P + g * LANES + lane
                n = base + rloc
                k3 = lax.rem(n, K) * 3
                b0 = plsc.load_gather(bary_v, [k3])
                b1 = plsc.load_gather(bary_v, [k3 + 1])
                b2 = plsc.load_gather(bary_v, [k3 + 2])
                r3 = rloc * 3
                p0 = plsc.load_gather(par_v, [r3]) * 2
                p1 = plsc.load_gather(par_v, [r3 + 1]) * 2
                p2 = plsc.load_gather(par_v, [r3 + 2]) * 2
                vx0 = plsc.load_gather(v2d_v, [p0])
                vy0 = plsc.load_gather(v2d_v, [p0 + 1])
                vx1 = plsc.load_gather(v2d_v, [p1])
                vy1 = plsc.load_gather(v2d_v, [p1 + 1])
                vx2 = plsc.load_gather(v2d_v, [p2])
                vy2 = plsc.load_gather(v2d_v, [p2 + 1])
                cx = vx0 * b0 + vx1 * b1 + vx2 * b2
                cy = vy0 * b0 + vy1 * b1 + vy2 * b2
                xn = cx / (W - 1) * 2.0 - 1.0
                yn = cy / (H - 1) * 2.0 - 1.0
                xf = (xn + 1.0) * 0.5 * (W - 1)
                yf = (yn + 1.0) * 0.5 * (H - 1)
                x0 = xf.astype(jnp.int32)
                y0 = yf.astype(jnp.int32)
                lx = xf - x0.astype(jnp.float32)
                ly = yf - y0.astype(jnp.float32)
                x1 = x0 + 1
                y1 = y0 + 1
                mx = x1 < W
                my = y1 < H
                w00 = (1.0 - lx) * (1.0 - ly)
                w01 = jnp.where(mx, lx * (1.0 - ly), 0.0)
                w10 = jnp.where(my, (1.0 - lx) * ly, 0.0)
                w11 = jnp.where(mx & my, lx * ly, 0.0)
                y1c = jnp.minimum(y1, H - 1)
                s = pl.ds(g * LANES, LANES)
                idx_v[b, 0, s] = y0 * W + x0
                idx_v[b, 1, s] = y1c * W + x0
                w_v[b, 0, s] = w00
                w_v[b, 1, s] = w01
                w_v[b, 2, s] = w10
                w_v[b, 3, s] = w11

        def issue_gathers(b):
            for t in range(2):
                pltpu.async_copy(table.at[idx_v.at[b, t]], rows_v.at[b, t],
                                 gsem[b])

        def drain_gathers(b):
            for t in range(2):
                pltpu.make_async_copy(table.at[idx_v.at[b, t]],
                                      rows_v.at[b, t], gsem[b]).wait()

        himask = jnp.full((LANES,), -65536, jnp.int32)

        def decode(v):
            lo = plsc.bitcast(lax.shift_left(v, 16), jnp.float32)
            hi = plsc.bitcast(v & himask, jnp.float32)
            return lo, hi

        Ch = C // 2

        def combine(ci, b):
            def grp_body(g2, c):
                b16 = g2 * LANES
                sw = pl.ds(b16, LANES)
                wv0 = w_v[b, 0, sw]
                wv1 = w_v[b, 1, sw]
                wv2 = w_v[b, 2, sw]
                wv3 = w_v[b, 3, sw]
                for ps in range(LANES):
                    p = b16 + ps
                    w00 = wv0[ps]
                    w01 = wv1[ps]
                    w10 = wv2[ps]
                    w11 = wv3[ps]
                    for cc in range(Ch // LANES):
                        sl0 = pl.ds(cc * LANES, LANES)
                        sl1 = pl.ds(Ch + cc * LANES, LANES)
                        lo00, hi00 = decode(rows_v[b, 0, p, sl0])
                        lo01, hi01 = decode(rows_v[b, 0, p, sl1])
                        lo10, hi10 = decode(rows_v[b, 1, p, sl0])
                        lo11, hi11 = decode(rows_v[b, 1, p, sl1])
                        accA = (lo00 * w00 + lo01 * w01
                                + lo10 * w10 + lo11 * w11)
                        accB = (hi00 * w00 + hi01 * w01
                                + hi10 * w10 + hi11 * w11)
                        out_v[b, p, sl0] = accA
                        out_v[b, p, sl1] = accB
                return c

            lax.fori_loop(0, P // LANES, grp_body, 0)

        def out_slice(ci):
            return out_hbm.at[pl.ds(base + ci * P, P)]

        stage_a(0, 0)
        issue_gathers(0)

        def outer(co, carry):
            for b in range(2):
                ci = co * 2 + b
                nb = 1 - b

                @pl.when(ci < nchunks - 1)
                def _():
                    stage_a(ci + 1, nb)
                    issue_gathers(nb)

                drain_gathers(b)

                @pl.when(co > 0)
                def _():
                    pltpu.make_async_copy(out_v.at[b], out_slice(ci - 2),
                                          osem[b]).wait()

                combine(ci, b)
                pltpu.make_async_copy(out_v.at[b], out_slice(ci),
                                      osem[b]).start()
            return carry

        lax.fori_loop(0, nchunks // 2, outer, 0)
        pltpu.make_async_copy(out_v.at[0], out_slice(nchunks - 2),
                              osem[0]).wait()
        pltpu.make_async_copy(out_v.at[1], out_slice(nchunks - 1),
                              osem[1]).wait()

    return sc_fn


def kernel(feature_map, vertices2d, bary, parents):
    B, C, H, W = feature_map.shape
    N = parents.shape[0]
    Nv = vertices2d.shape[0]
    K = bary.shape[0]
    table = _transpose_call(C, H, W, 8)(feature_map)
    out = _sc_sample_call(N, C, H, W, Nv, K)(
        table, parents.reshape(-1), bary.reshape(-1), vertices2d.reshape(-1))
    return out.reshape(B, N, C)


# speedup vs baseline: 1.6591x; 1.0119x over previous
import functools

import jax
import jax.numpy as jnp
from jax import lax
from jax.experimental import pallas as pl
from jax.experimental.pallas import tpu as pltpu
from jax.experimental.pallas import tpu_sc as plsc

NW = 32
P = 64
LANES = 16


@functools.lru_cache(maxsize=None)
def _transpose_call(C, H, W, HB):
    Ch = C // 2

    def body(in_ref, out_ref):
        blk = in_ref[0]
        t = jnp.reshape(blk, (C, HB * W)).T
        u = lax.bitcast_convert_type(t, jnp.int32)

        def rnd(v):
            return v + 0x7FFF + (lax.shift_right_logical(v, 16) & 1)

        word = (lax.shift_right_logical(rnd(u[:, :Ch]), 16)
                | (rnd(u[:, Ch:]) & jnp.int32(-65536)))
        out_ref[:, :Ch] = word
        out_ref[:, Ch:] = jnp.roll(word, -1, axis=0)

    return pl.pallas_call(
        body,
        grid=(H // HB,),
        in_specs=[pl.BlockSpec((1, C, HB, W), lambda i: (0, 0, i, 0))],
        out_specs=pl.BlockSpec((HB * W, C), lambda i: (i, 0)),
        out_shape=jax.ShapeDtypeStruct((H * W, C), jnp.int32),
    )


@functools.lru_cache(maxsize=None)
def _sc_sample_call(N, C, H, W, Nv, K):
    rows_per_tile = N // NW
    nchunks = rows_per_tile // P
    mesh = plsc.VectorSubcoreMesh(core_axis_name="c", subcore_axis_name="s")

    @functools.partial(
        pl.kernel,
        mesh=mesh,
        out_type=jax.ShapeDtypeStruct((N, C), jnp.float32),
        compiler_params=pltpu.CompilerParams(needs_layout_passes=False),
        scratch_types=[
            pltpu.VMEM((Nv * 2,), jnp.float32),
            pltpu.VMEM((K * 3,), jnp.float32),
            pltpu.VMEM((rows_per_tile * 3,), jnp.int32),
            pltpu.VMEM((2, 2, P), jnp.int32),
            pltpu.VMEM((2, 4, P), jnp.float32),
            pltpu.VMEM((2, 2, P, C), jnp.int32),
            pltpu.VMEM((2, P, C), jnp.float32),
            pltpu.SemaphoreType.DMA,
            pltpu.SemaphoreType.DMA,
            pltpu.SemaphoreType.DMA,
            pltpu.SemaphoreType.DMA,
        ],
    )
    def sc_fn(table, parents_hbm, bary_hbm, v2d_hbm, out_hbm,
              v2d_v, bary_v, par_v, idx_v, w_v, rows_v, out_v,
              gsem0, gsem1, osem0, osem1):
        cid = lax.axis_index("c")
        sid = lax.axis_index("s")
        wid = sid * 2 + cid
        base = wid * rows_per_tile
        gsem = (gsem0, gsem1)
        osem = (osem0, osem1)

        pltpu.sync_copy(v2d_hbm, v2d_v)
        pltpu.sync_copy(bary_hbm, bary_v)
        pltpu.sync_copy(parents_hbm.at[pl.ds(base * 3, rows_per_tile * 3)],
                        par_v)

        lane = lax.iota(jnp.int32, LANES)

        def stage_a(ci, b):
            for g in range(P // LANES):
                rloc = ci * P + g * LANES + lane
                n = base + rloc
                k3 = lax.rem(n, K) * 3
                b0 = plsc.load_gather(bary_v, [k3])
                b1 = plsc.load_gather(bary_v, [k3 + 1])
                b2 = plsc.load_gather(bary_v, [k3 + 2])
                r3 = rloc * 3
                p0 = plsc.load_gather(par_v, [r3]) * 2
                p1 = plsc.load_gather(par_v, [r3 + 1]) * 2
                p2 = plsc.load_gather(par_v, [r3 + 2]) * 2
                vx0 = plsc.load_gather(v2d_v, [p0])
                vy0 = plsc.load_gather(v2d_v, [p0 + 1])
                vx1 = plsc.load_gather(v2d_v, [p1])
                vy1 = plsc.load_gather(v2d_v, [p1 + 1])
                vx2 = plsc.load_gather(v2d_v, [p2])
                vy2 = plsc.load_gather(v2d_v, [p2 + 1])
                cx = vx0 * b0 + vx1 * b1 + vx2 * b2
                cy = vy0 * b0 + vy1 * b1 + vy2 * b2
                xn = cx / (W - 1) * 2.0 - 1.0
                yn = cy / (H - 1) * 2.0 - 1.0
                xf = (xn + 1.0) * 0.5 * (W - 1)
                yf = (yn + 1.0) * 0.5 * (H - 1)
                x0 = xf.astype(jnp.int32)
                y0 = yf.astype(jnp.int32)
                lx = xf - x0.astype(jnp.float32)
                ly = yf - y0.astype(jnp.float32)
                x1 = x0 + 1
                y1 = y0 + 1
                mx = x1 < W
                my = y1 < H
                w00 = (1.0 - lx) * (1.0 - ly)
                w01 = jnp.where(mx, lx * (1.0 - ly), 0.0)
                w10 = jnp.where(my, (1.0 - lx) * ly, 0.0)
                w11 = jnp.where(mx & my, lx * ly, 0.0)
                y1c = jnp.minimum(y1, H - 1)
                s = pl.ds(g * LANES, LANES)
                idx_v[b, 0, s] = y0 * W + x0
                idx_v[b, 1, s] = y1c * W + x0
                w_v[b, 0, s] = w00
                w_v[b, 1, s] = w01
                w_v[b, 2, s] = w10
                w_v[b, 3, s] = w11

        def issue_gathers(b):
            for t in range(2):
                pltpu.async_copy(table.at[idx_v.at[b, t]], rows_v.at[b, t],
                                 gsem[b])

        def drain_gathers(b):
            for t in range(2):
                pltpu.make_async_copy(table.at[idx_v.at[b, t]],
                                      rows_v.at[b, t], gsem[b]).wait()

        himask = jnp.full((LANES,), -65536, jnp.int32)

        def decode(v):
            lo = plsc.bitcast(lax.shift_left(v, 16), jnp.float32)
            hi = plsc.bitcast(v & himask, jnp.float32)
            return lo, hi

        Ch = C // 2

        def combine(ci, b):
            def grp_body(g2, c):
                b16 = g2 * LANES
                sw = pl.ds(b16, LANES)
                wv0 = w_v[b, 0, sw]
                wv1 = w_v[b, 1, sw]
                wv2 = w_v[b, 2, sw]
                wv3 = w_v[b, 3, sw]
                for ps in range(LANES):
                    p = b16 + ps
                    w00 = wv0[ps]
                    w01 = wv1[ps]
                    w10 = wv2[ps]
                    w11 = wv3[ps]
                    for cc in range(Ch // LANES):
                        sl0 = pl.ds(cc * LANES, LANES)
                        sl1 = pl.ds(Ch + cc * LANES, LANES)
                        lo00, hi00 = decode(rows_v[b, 0, p, sl0])
                        lo01, hi01 = decode(rows_v[b, 0, p, sl1])
                        lo10, hi10 = decode(rows_v[b, 1, p, sl0])
                        lo11, hi11 = decode(rows_v[b, 1, p, sl1])
                        accA = (lo00 * w00 + lo01 * w01
                                + lo10 * w10 + lo11 * w11)
                        accB = (hi00 * w00 + hi01 * w01
                                + hi10 * w10 + hi11 * w11)
                        out_v[b, p, sl0] = accA
                        out_v[b, p, sl1] = accB
                return c

            lax.fori_loop(0, P // LANES, grp_body, 0)

        def out_slice(ci):
            return out_hbm.at[pl.ds(base + ci * P, P)]

        stage_a(0, 0)
        issue_gathers(0)

        def outer(co, carry):
            for b in range(2):
                ci = co * 2 + b
                nb = 1 - b

                @pl.when(ci < nchunks - 1)
                def _():
                    stage_a(ci + 1, nb)
                    issue_gathers(nb)

                drain_gathers(b)

                @pl.when(co > 0)
                def _():
                    pltpu.make_async_copy(out_v.at[b], out_slice(ci - 2),
                                          osem[b]).wait()

                combine(ci, b)
                pltpu.make_async_copy(out_v.at[b], out_slice(ci),
                                      osem[b]).start()
            return carry

        lax.fori_loop(0, nchunks // 2, outer, 0)
        pltpu.make_async_copy(out_v.at[0], out_slice(nchunks - 2),
                              osem[0]).wait()
        pltpu.make_async_copy(out_v.at[1], out_slice(nchunks - 1),
                              osem[1]).wait()

    return sc_fn


def kernel(feature_map, vertices2d, bary, parents):
    B, C, H, W = feature_map.shape
    N = parents.shape[0]
    Nv = vertices2d.shape[0]
    K = bary.shape[0]
    table = _transpose_call(C, H, W, 16)(feature_map)
    out = _sc_sample_call(N, C, H, W, Nv, K)(
        table, parents.reshape(-1), bary.reshape(-1), vertices2d.reshape(-1))
    return out.reshape(B, N, C)
